# Initial kernel scaffold; baseline (speedup 1.0000x reference)
import jax
import jax.numpy as jnp
from jax.experimental import pallas as pl


def kernel(input_ids, token_embedding, position_embedding):
    raise NotImplementedError("write your pallas kernel here")


# speedup vs baseline: 3.0740x; 3.0740x over previous
import functools

import jax
import jax.numpy as jnp
from jax import lax
from jax.experimental import pallas as pl
from jax.experimental.pallas import tpu as pltpu
from jax.experimental.pallas import tpu_sc as plsc

_EMBED = 128
_SEQ = 50
_NUM_WORKERS = 32
_SPC = 2
_LANES = 16
_VPR = _EMBED // _LANES


def _sc_lookup(ids, table, pos):
    batch = ids.shape[0]
    seqs_per_w = batch // _NUM_WORKERS
    n_chunks = seqs_per_w // _SPC
    mesh = plsc.VectorSubcoreMesh(core_axis_name="c", subcore_axis_name="s")

    @functools.partial(
        pl.kernel,
        out_type=jax.ShapeDtypeStruct((batch, _SEQ, _EMBED), jnp.float32),
        mesh=mesh,
        compiler_params=pltpu.CompilerParams(use_tc_tiling_on_sc=False),
        scratch_types=[
            pltpu.VMEM((seqs_per_w, _SEQ), jnp.int32),
            pltpu.VMEM((_SEQ, _EMBED), jnp.float32),
            pltpu.VMEM((2, _SPC, _SEQ, _EMBED), jnp.float32),
            pltpu.SemaphoreType.DMA,
            pltpu.SemaphoreType.DMA,
            pltpu.SemaphoreType.DMA,
            pltpu.SemaphoreType.DMA,
        ],
    )
    def body(ids_hbm, table_hbm, pos_hbm, out_hbm, idx_v, pos_v, rows_v,
             g_sem0, g_sem1, s_sem0, s_sem1):
        nc = plsc.get_sparse_core_info().num_cores
        wid = lax.axis_index("s") * nc + lax.axis_index("c")
        sbase = wid * seqs_per_w
        g_sems = (g_sem0, g_sem1)
        s_sems = (s_sem0, s_sem1)

        pltpu.sync_copy(ids_hbm.at[pl.ds(sbase, seqs_per_w)], idx_v)
        pltpu.sync_copy(pos_hbm.at[pl.ds(0, _SEQ)], pos_v)

        def start_gather(j, buf):
            for s in range(_SPC):
                pltpu.async_copy(table_hbm.at[idx_v.at[j * _SPC + s]],
                                 rows_v.at[buf, s], g_sems[buf])

        def wait_gather(buf):
            for s in range(_SPC):
                pltpu.make_async_copy(table_hbm.at[idx_v.at[0]],
                                      rows_v.at[buf, s], g_sems[buf]).wait()

        def start_store(j, buf):
            pltpu.async_copy(rows_v.at[buf],
                             out_hbm.at[pl.ds(sbase + j * _SPC, _SPC)],
                             s_sems[buf])

        def wait_store(buf):
            pltpu.make_async_copy(rows_v.at[buf],
                                  out_hbm.at[pl.ds(sbase, _SPC)],
                                  s_sems[buf]).wait()

        def add_pos(buf):
            def row_add(l, _):
                for k in range(_VPR):
                    sl = pl.ds(k * _LANES, _LANES)
                    p = pos_v[l, sl]
                    for s in range(_SPC):
                        rows_v[buf, s, l, sl] = rows_v[buf, s, l, sl] + p
                return 0
            lax.fori_loop(0, _SEQ, row_add, 0, unroll=False)

        start_gather(0, 0)
        wait_gather(0)
        start_gather(1, 1)
        add_pos(0)
        start_store(0, 0)

        def two_chunks(i, _):
            for b in (1, 0):
                j = 2 * i + (1 if b == 1 else 2)
                wait_gather(b)
                other = 1 - b
                wait_store(other)
                start_gather(j + 1, other)
                add_pos(b)
                start_store(j, b)
            return 0

        lax.fori_loop(0, (n_chunks - 2) // 2, two_chunks, 0, unroll=False)

        wait_gather(1)
        add_pos(1)
        start_store(n_chunks - 1, 1)
        wait_store(0)
        wait_store(1)

    return body(ids, table, pos)


def kernel(input_ids, token_embedding, position_embedding):
    return _sc_lookup(input_ids.astype(jnp.int32), token_embedding,
                      position_embedding)


# speedup vs baseline: 5.1668x; 1.6808x over previous
import functools

import jax
import jax.numpy as jnp
from jax import lax
from jax.experimental import pallas as pl
from jax.experimental.pallas import tpu as pltpu
from jax.experimental.pallas import tpu_sc as plsc

_EMBED = 128
_SEQ = 50
_NUM_WORKERS = 32
_SPC = 2
_LANES = 16
_VPR = _EMBED // _LANES


def _sc_lookup(ids, table, pos):
    batch = ids.shape[0]
    seqs_per_w = batch // _NUM_WORKERS
    n_chunks = seqs_per_w // _SPC
    mesh = plsc.VectorSubcoreMesh(core_axis_name="c", subcore_axis_name="s")

    @functools.partial(
        pl.kernel,
        out_type=jax.ShapeDtypeStruct((batch, _SEQ, _EMBED), jnp.float32),
        mesh=mesh,
        scratch_types=[
            pltpu.VMEM((seqs_per_w, _SEQ), jnp.int32),
            pltpu.VMEM((56, _EMBED), jnp.float32),
            pltpu.VMEM((2, _SPC, _SEQ, _EMBED), jnp.float32),
            pltpu.SemaphoreType.DMA,
            pltpu.SemaphoreType.DMA,
            pltpu.SemaphoreType.DMA,
            pltpu.SemaphoreType.DMA,
        ],
    )
    def body(ids_hbm, table_hbm, pos_hbm, out_hbm, idx_v, pos_v, rows_v,
             g_sem0, g_sem1, s_sem0, s_sem1):
        nc = plsc.get_sparse_core_info().num_cores
        wid = lax.axis_index("s") * nc + lax.axis_index("c")
        sbase = wid * seqs_per_w
        g_sems = (g_sem0, g_sem1)
        s_sems = (s_sem0, s_sem1)

        pltpu.sync_copy(ids_hbm.at[pl.ds(sbase, seqs_per_w)], idx_v)
        pltpu.sync_copy(pos_hbm.at[pl.ds(0, 56)], pos_v)

        def start_gather(j, buf):
            for s in range(_SPC):
                pltpu.async_copy(table_hbm.at[idx_v.at[j * _SPC + s]],
                                 rows_v.at[buf, s], g_sems[buf])

        def wait_gather(buf):
            for s in range(_SPC):
                pltpu.make_async_copy(table_hbm.at[idx_v.at[0]],
                                      rows_v.at[buf, s], g_sems[buf]).wait()

        def start_store(j, buf):
            pltpu.async_copy(rows_v.at[buf],
                             out_hbm.at[pl.ds(sbase + j * _SPC, _SPC)],
                             s_sems[buf])

        def wait_store(buf):
            pltpu.make_async_copy(rows_v.at[buf],
                                  out_hbm.at[pl.ds(sbase, _SPC)],
                                  s_sems[buf]).wait()

        def add_pos(buf):
            def row_add(l, _):
                for k in range(_VPR):
                    sl = pl.ds(k * _LANES, _LANES)
                    p = pos_v[l, sl]
                    for s in range(_SPC):
                        rows_v[buf, s, l, sl] = rows_v[buf, s, l, sl] + p
                return 0
            lax.fori_loop(0, _SEQ, row_add, 0, unroll=False)

        start_gather(0, 0)
        wait_gather(0)
        start_gather(1, 1)
        add_pos(0)
        start_store(0, 0)

        def two_chunks(i, _):
            for b in (1, 0):
                j = 2 * i + (1 if b == 1 else 2)
                wait_gather(b)
                other = 1 - b
                wait_store(other)
                start_gather(j + 1, other)
                add_pos(b)
                start_store(j, b)
            return 0

        lax.fori_loop(0, (n_chunks - 2) // 2, two_chunks, 0, unroll=False)

        wait_gather(1)
        add_pos(1)
        start_store(n_chunks - 1, 1)
        wait_store(0)
        wait_store(1)

    return body(ids, table, pos)


def kernel(input_ids, token_embedding, position_embedding):
    return _sc_lookup(input_ids.astype(jnp.int32), token_embedding,
                      position_embedding)


# speedup vs baseline: 8.9238x; 1.7271x over previous
import functools

import jax
import jax.numpy as jnp
from jax import lax
from jax.experimental import pallas as pl
from jax.experimental.pallas import tpu as pltpu
from jax.experimental.pallas import tpu_sc as plsc

_EMBED = 128
_NUM_WORKERS = 32
_LANES = 16
_VPR = _EMBED // _LANES
_POS_PAD = 56


def _sc_lookup(ids_t, table, pos):
    seq, batch = ids_t.shape
    bpw = batch // _NUM_WORKERS
    mesh = plsc.VectorSubcoreMesh(core_axis_name="c", subcore_axis_name="s")

    @functools.partial(
        pl.kernel,
        out_type=jax.ShapeDtypeStruct((seq, batch, _EMBED), jnp.float32),
        mesh=mesh,
        scratch_types=[
            pltpu.VMEM((seq, bpw), jnp.int32),
            pltpu.VMEM((_POS_PAD, _EMBED), jnp.float32),
            pltpu.VMEM((2, bpw, _EMBED), jnp.float32),
            pltpu.SemaphoreType.DMA,
            pltpu.SemaphoreType.DMA,
            pltpu.SemaphoreType.DMA,
            pltpu.SemaphoreType.DMA,
        ],
    )
    def body(ids_hbm, table_hbm, pos_hbm, out_hbm, idx_v, pos_v, rows_v,
             g_sem0, g_sem1, s_sem0, s_sem1):
        nc = plsc.get_sparse_core_info().num_cores
        wid = lax.axis_index("s") * nc + lax.axis_index("c")
        b0 = wid * bpw
        g_sems = (g_sem0, g_sem1)
        s_sems = (s_sem0, s_sem1)

        pltpu.sync_copy(ids_hbm.at[:, pl.ds(b0, bpw)], idx_v)
        pltpu.sync_copy(pos_hbm.at[pl.ds(0, _POS_PAD)], pos_v)

        def start_gather(l, buf):
            pltpu.async_copy(table_hbm.at[idx_v.at[l]], rows_v.at[buf],
                             g_sems[buf])

        def wait_gather(buf):
            pltpu.make_async_copy(table_hbm.at[idx_v.at[0]], rows_v.at[buf],
                                  g_sems[buf]).wait()

        def start_store(l, buf):
            pltpu.async_copy(rows_v.at[buf], out_hbm.at[l, pl.ds(b0, bpw)],
                             s_sems[buf])

        def wait_store(buf):
            pltpu.make_async_copy(rows_v.at[buf], out_hbm.at[0, pl.ds(b0, bpw)],
                                  s_sems[buf]).wait()

        def add_pos(l, buf):
            p = [pos_v[l, pl.ds(k * _LANES, _LANES)] for k in range(_VPR)]

            def row_add(r, _):
                for k in range(_VPR):
                    sl = pl.ds(k * _LANES, _LANES)
                    rows_v[buf, r, sl] = rows_v[buf, r, sl] + p[k]
                return 0
            lax.fori_loop(0, bpw, row_add, 0, unroll=False)

        start_gather(0, 0)
        wait_gather(0)
        start_gather(1, 1)
        add_pos(0, 0)
        start_store(0, 0)

        def two_items(i, _):
            for b in (1, 0):
                l = 2 * i + (1 if b == 1 else 2)
                wait_gather(b)
                other = 1 - b
                wait_store(other)
                start_gather(l + 1, other)
                add_pos(l, b)
                start_store(l, b)
            return 0

        lax.fori_loop(0, (seq - 2) // 2, two_items, 0, unroll=False)

        wait_gather(1)
        add_pos(seq - 1, 1)
        start_store(seq - 1, 1)
        wait_store(0)
        wait_store(1)

    return body(ids_t, table, pos)


def kernel(input_ids, token_embedding, position_embedding):
    ids_t = jnp.transpose(input_ids.astype(jnp.int32))
    out_t = _sc_lookup(ids_t, token_embedding, position_embedding)
    return jnp.transpose(out_t, (1, 0, 2))


# speedup vs baseline: 10.3112x; 1.1555x over previous
import functools

import jax
import jax.numpy as jnp
from jax import lax
from jax.experimental import pallas as pl
from jax.experimental.pallas import tpu as pltpu
from jax.experimental.pallas import tpu_sc as plsc

_EMBED = 128
_NUM_WORKERS = 32
_LANES = 16
_VPR = _EMBED // _LANES
_POS_PAD = 56


def _sc_lookup(ids_t, table, pos):
    seq, batch = ids_t.shape
    bpw = batch // _NUM_WORKERS
    mesh = plsc.VectorSubcoreMesh(core_axis_name="c", subcore_axis_name="s")

    @functools.partial(
        pl.kernel,
        out_type=jax.ShapeDtypeStruct((seq, batch, _EMBED), jnp.float32),
        mesh=mesh,
        scratch_types=[
            pltpu.VMEM((seq, bpw), jnp.int32),
            pltpu.VMEM((_POS_PAD, _EMBED), jnp.float32),
            pltpu.VMEM((4, bpw, _EMBED), jnp.float32),
            pltpu.SemaphoreType.DMA,
            pltpu.SemaphoreType.DMA,
            pltpu.SemaphoreType.DMA,
            pltpu.SemaphoreType.DMA,
            pltpu.SemaphoreType.DMA,
            pltpu.SemaphoreType.DMA,
            pltpu.SemaphoreType.DMA,
            pltpu.SemaphoreType.DMA,
        ],
    )
    def body(ids_hbm, table_hbm, pos_hbm, out_hbm, idx_v, pos_v, rows_v,
             g_sem0, g_sem1, g_sem2, g_sem3, s_sem0, s_sem1, s_sem2, s_sem3):
        nc = plsc.get_sparse_core_info().num_cores
        wid = lax.axis_index("s") * nc + lax.axis_index("c")
        b0 = wid * bpw
        g_sems = (g_sem0, g_sem1, g_sem2, g_sem3)
        s_sems = (s_sem0, s_sem1, s_sem2, s_sem3)

        pltpu.sync_copy(ids_hbm.at[:, pl.ds(b0, bpw)], idx_v)
        pltpu.sync_copy(pos_hbm.at[pl.ds(0, _POS_PAD)], pos_v)

        def start_gather(l, buf):
            pltpu.async_copy(table_hbm.at[idx_v.at[l]], rows_v.at[buf],
                             g_sems[buf])

        def wait_gather(buf):
            pltpu.make_async_copy(table_hbm.at[idx_v.at[0]], rows_v.at[buf],
                                  g_sems[buf]).wait()

        def start_store(l, buf):
            pltpu.async_copy(rows_v.at[buf], out_hbm.at[l, pl.ds(b0, bpw)],
                             s_sems[buf])

        def wait_store(buf):
            pltpu.make_async_copy(rows_v.at[buf], out_hbm.at[0, pl.ds(b0, bpw)],
                                  s_sems[buf]).wait()

        def add_pos(l, buf):
            p = [pos_v[l, pl.ds(k * _LANES, _LANES)] for k in range(_VPR)]

            def row_add(r, _):
                for k in range(_VPR):
                    sl = pl.ds(k * _LANES, _LANES)
                    rows_v[buf, r, sl] = rows_v[buf, r, sl] + p[k]
                return 0
            lax.fori_loop(0, bpw, row_add, 0, unroll=False)

        start_gather(0, 0)
        start_gather(1, 1)

        def process(l, b, pre_b, wait_prev_store):
            if wait_prev_store:
                wait_store(pre_b)
            start_gather(l + 2, pre_b)
            wait_gather(b)
            add_pos(l, b)
            start_store(l, b)

        process(0, 0, 2, False)
        process(1, 1, 3, False)
        process(2, 2, 0, True)
        process(3, 3, 1, True)

        def four_items(i, _):
            for b in range(4):
                l = 4 * i + 4 + b
                process(l, b, (b + 2) % 4, True)
            return 0

        lax.fori_loop(0, (seq - 6) // 4, four_items, 0, unroll=False)

        for l in (seq - 2, seq - 1):
            b = l % 4
            wait_gather(b)
            add_pos(l, b)
            start_store(l, b)
        for b in range(4):
            wait_store(b)

    return body(ids_t, table, pos)


def kernel(input_ids, token_embedding, position_embedding):
    ids_t = jnp.transpose(input_ids.astype(jnp.int32))
    out_t = _sc_lookup(ids_t, token_embedding, position_embedding)
    return jnp.transpose(out_t, (1, 0, 2))
